# baseline (speedup 1.0000x reference)
import functools

import jax
import jax.numpy as jnp
from jax import lax
from jax.experimental import pallas as pl
from jax.experimental.pallas import tpu as pltpu
from jax.experimental.pallas import tpu_sc as plsc

_K = 1024
_D = 64
_BETA = 0.25
_R = 256
_KC = 256

_NC = 2
_NS = 16
_NW = _NC * _NS
_CH = 128
_NBUF = 3


def _argmin_body(x2_ref, xsq_ref, w_ref, wsq_ref, kio_ref, idx_ref, loss_ref,
                 acc_ref, *, n_tokens):
    x2 = x2_ref[...] * 2.0
    xsq_row = xsq_ref[...][None, :]

    big = float(2 * _K)
    run_min8 = jnp.full((8, _R), jnp.inf, dtype=jnp.float32)
    run_k8 = jnp.zeros((8, _R), dtype=jnp.float32)
    for c in range(_K // _KC):
        wc = w_ref[c * _KC:(c + 1) * _KC, :]
        wsq_c = wsq_ref[c * _KC:(c + 1) * _KC, :]
        mm2 = lax.dot_general(wc, x2, (((1,), (1,)), ((), ())),
                              preferred_element_type=jnp.float32)
        dist = (xsq_row + wsq_c) - mm2
        d3 = dist.reshape(_KC // 8, 8, _R)
        cmin8 = jnp.min(d3, axis=0)
        kio_c = kio_ref[c * (_KC // 8):(c + 1) * (_KC // 8), :][:, :, None]
        ck8 = jnp.min(jnp.where(d3 == cmin8[None], kio_c, big), axis=0)
        better = cmin8 < run_min8
        run_k8 = jnp.where(better, ck8, run_k8)
        run_min8 = jnp.where(better, cmin8, run_min8)

    gmin = jnp.min(run_min8, axis=0)
    kbest = jnp.min(jnp.where(run_min8 == gmin[None, :], run_k8, big), axis=0)
    idx_ref[...] = kbest.astype(jnp.int32)

    @pl.when(pl.program_id(0) == 0)
    def _init():
        acc_ref[0, 0] = 0.0

    acc_ref[0, 0] += jnp.sum(gmin)
    scale = (1.0 + _BETA) / (n_tokens * _D)
    loss_ref[...] = jnp.broadcast_to(acc_ref[0, 0] * scale, (1, 1))


def _gather_body(w_hbm, idx_hbm, out_hbm, idx_v, rows, gsems, wsems, *, chunks):
    wid = lax.axis_index("s") * _NC + lax.axis_index("c")
    base = wid * (chunks * _CH)
    pltpu.sync_copy(idx_hbm.at[pl.ds(base, chunks * _CH)], idx_v)
    gcps = [None] * _NBUF
    wcps = [None] * _NBUF
    for j in range(chunks):
        b = j % _NBUF
        if wcps[b] is not None:
            wcps[b].wait()
        gcps[b] = pltpu.async_copy(
            w_hbm.at[idx_v.at[pl.ds(j * _CH, _CH)]], rows[b], gsems[b])
        if j >= _NBUF - 1:
            jw = j - (_NBUF - 1)
            bw = jw % _NBUF
            gcps[bw].wait()
            wcps[bw] = pltpu.async_copy(
                rows[bw], out_hbm.at[pl.ds(base + jw * _CH, _CH)], wsems[bw])
    for jw in range(max(chunks - _NBUF + 1, 0), chunks):
        bw = jw % _NBUF
        gcps[bw].wait()
        wcps[bw] = pltpu.async_copy(
            rows[bw], out_hbm.at[pl.ds(base + jw * _CH, _CH)], wsems[bw])
    for b in range(_NBUF):
        if wcps[b] is not None:
            wcps[b].wait()


def kernel(latents, W):
    lat = latents.reshape(-1, latents.shape[-2], _D)
    flat = lat.reshape(-1, _D)
    n = flat.shape[0]
    xsq = jnp.sum(flat ** 2, axis=1)
    wsq = jnp.sum(W ** 2, axis=1)[:, None]
    kio = jnp.arange(_K, dtype=jnp.float32).reshape(_K // 8, 8)

    grid = (n // _R,)
    idx, loss = pl.pallas_call(
        functools.partial(_argmin_body, n_tokens=n),
        grid=grid,
        in_specs=[
            pl.BlockSpec((_R, _D), lambda i: (i, 0)),
            pl.BlockSpec((_R,), lambda i: (i,)),
            pl.BlockSpec((_K, _D), lambda i: (0, 0)),
            pl.BlockSpec((_K, 1), lambda i: (0, 0)),
            pl.BlockSpec((_K // 8, 8), lambda i: (0, 0)),
        ],
        out_specs=[
            pl.BlockSpec((_R,), lambda i: (i,)),
            pl.BlockSpec((1, 1), lambda i: (0, 0)),
        ],
        out_shape=[
            jax.ShapeDtypeStruct((n,), jnp.int32),
            jax.ShapeDtypeStruct((1, 1), jnp.float32),
        ],
        scratch_shapes=[pltpu.SMEM((1, 1), jnp.float32)],
    )(flat, xsq, W, wsq, kio)

    chunks = n // (_NW * _CH)
    mesh = plsc.VectorSubcoreMesh(core_axis_name="c", subcore_axis_name="s")
    q = pl.kernel(
        functools.partial(_gather_body, chunks=chunks),
        mesh=mesh,
        out_type=jax.ShapeDtypeStruct((n, _D), jnp.float32),
        scratch_types=[
            pltpu.VMEM((chunks * _CH,), jnp.int32),
            [pltpu.VMEM((_CH, _D), jnp.float32) for _ in range(_NBUF)],
            [pltpu.SemaphoreType.DMA for _ in range(_NBUF)],
            [pltpu.SemaphoreType.DMA for _ in range(_NBUF)],
        ],
        compiler_params=pltpu.CompilerParams(use_tc_tiling_on_sc=False),
    )(W, idx)

    quantized_st = q.reshape(lat.shape)
    vq_loss = loss[0, 0]
    return (quantized_st, vq_loss)


# speedup vs baseline: 1.3156x; 1.3156x over previous
import functools

import jax
import jax.numpy as jnp
from jax import lax
from jax.experimental import pallas as pl
from jax.experimental.pallas import tpu as pltpu
from jax.experimental.pallas import tpu_sc as plsc

_K = 1024
_D = 64
_BETA = 0.25
_KC = 256

_NC = 2
_NS = 16
_NW = _NC * _NS
_CH = 128
_NBUF = 3


def _argmin_body(xt_ref, xsq_ref, w_ref, wsq_ref, kio_ref, idx_ref, loss_ref,
                 acc_ref, *, n_tokens, r):
    x2t = xt_ref[0] * 2.0
    xsq_row = xsq_ref[0]

    big = float(2 * _K)
    run_min8 = jnp.full((8, r), jnp.inf, dtype=jnp.float32)
    run_k8 = jnp.zeros((8, r), dtype=jnp.float32)
    for c in range(_K // _KC):
        wc = w_ref[c * _KC:(c + 1) * _KC, :]
        wsq_c = wsq_ref[c * _KC:(c + 1) * _KC, :]
        mm2 = lax.dot_general(wc, x2t, (((1,), (0,)), ((), ())),
                              preferred_element_type=jnp.float32)
        dist = (xsq_row + wsq_c) - mm2
        d3 = dist.reshape(_KC // 8, 8, r)
        cmin8 = jnp.min(d3, axis=0)
        kio_c = kio_ref[c * (_KC // 8):(c + 1) * (_KC // 8), :][:, :, None]
        ck8 = jnp.min(jnp.where(d3 == cmin8[None], kio_c, big), axis=0)
        better = cmin8 < run_min8
        run_k8 = jnp.where(better, ck8, run_k8)
        run_min8 = jnp.where(better, cmin8, run_min8)

    gmin = jnp.min(run_min8, axis=0)
    kbest = jnp.min(jnp.where(run_min8 == gmin[None, :], run_k8, big), axis=0)
    idx_ref[...] = kbest.astype(jnp.int32)[None, None, :]

    @pl.when(pl.program_id(0) == 0)
    def _init():
        acc_ref[0, 0] = 0.0

    acc_ref[0, 0] += jnp.sum(gmin)
    scale = (1.0 + _BETA) / (n_tokens * _D)
    loss_ref[...] = jnp.broadcast_to(acc_ref[0, 0] * scale, (1, 1))


def _gather_body(w_hbm, idx_hbm, out_hbm, idx_v, rows, gsems, wsems, *, chunks):
    wid = lax.axis_index("s") * _NC + lax.axis_index("c")
    base = wid * (chunks * _CH)
    pltpu.sync_copy(idx_hbm.at[pl.ds(base, chunks * _CH)], idx_v)
    gcps = [None] * _NBUF
    wcps = [None] * _NBUF
    for j in range(chunks):
        b = j % _NBUF
        if wcps[b] is not None:
            wcps[b].wait()
        gcps[b] = pltpu.async_copy(
            w_hbm.at[idx_v.at[pl.ds(j * _CH, _CH)]], rows[b], gsems[b])
        if j >= _NBUF - 1:
            jw = j - (_NBUF - 1)
            bw = jw % _NBUF
            gcps[bw].wait()
            wcps[bw] = pltpu.async_copy(
                rows[bw], out_hbm.at[pl.ds(base + jw * _CH, _CH)], wsems[bw])
    for jw in range(max(chunks - _NBUF + 1, 0), chunks):
        bw = jw % _NBUF
        gcps[bw].wait()
        wcps[bw] = pltpu.async_copy(
            rows[bw], out_hbm.at[pl.ds(base + jw * _CH, _CH)], wsems[bw])
    for b in range(_NBUF):
        if wcps[b] is not None:
            wcps[b].wait()


def kernel(latents, W):
    lat = latents.reshape(-1, latents.shape[-2], _D)
    flat = lat.reshape(-1, _D)
    n = flat.shape[0]
    nb, node = lat.shape[0], lat.shape[1]
    latT = jnp.transpose(lat, (0, 2, 1))
    xsq = jnp.sum(flat ** 2, axis=1).reshape(nb, 1, node)
    wsq = jnp.sum(W ** 2, axis=1)[:, None]
    kio = jnp.arange(_K, dtype=jnp.float32).reshape(_K // 8, 8)

    grid = (nb,)
    idx, loss = pl.pallas_call(
        functools.partial(_argmin_body, n_tokens=n, r=node),
        grid=grid,
        in_specs=[
            pl.BlockSpec((1, _D, node), lambda i: (i, 0, 0)),
            pl.BlockSpec((1, 1, node), lambda i: (i, 0, 0)),
            pl.BlockSpec((_K, _D), lambda i: (0, 0)),
            pl.BlockSpec((_K, 1), lambda i: (0, 0)),
            pl.BlockSpec((_K // 8, 8), lambda i: (0, 0)),
        ],
        out_specs=[
            pl.BlockSpec((1, 1, node), lambda i: (i, 0, 0)),
            pl.BlockSpec((1, 1), lambda i: (0, 0)),
        ],
        out_shape=[
            jax.ShapeDtypeStruct((nb, 1, node), jnp.int32),
            jax.ShapeDtypeStruct((1, 1), jnp.float32),
        ],
        scratch_shapes=[pltpu.SMEM((1, 1), jnp.float32)],
    )(latT, xsq, W, wsq, kio)
    idx = idx.reshape(n)

    chunks = n // (_NW * _CH)
    mesh = plsc.VectorSubcoreMesh(core_axis_name="c", subcore_axis_name="s")
    q = pl.kernel(
        functools.partial(_gather_body, chunks=chunks),
        mesh=mesh,
        out_type=jax.ShapeDtypeStruct((n, _D), jnp.float32),
        scratch_types=[
            pltpu.VMEM((chunks * _CH,), jnp.int32),
            [pltpu.VMEM((_CH, _D), jnp.float32) for _ in range(_NBUF)],
            [pltpu.SemaphoreType.DMA for _ in range(_NBUF)],
            [pltpu.SemaphoreType.DMA for _ in range(_NBUF)],
        ],
        compiler_params=pltpu.CompilerParams(use_tc_tiling_on_sc=False),
    )(W, idx)

    quantized_st = q.reshape(lat.shape)
    vq_loss = loss[0, 0]
    return (quantized_st, vq_loss)


# speedup vs baseline: 1.8871x; 1.4344x over previous
import functools

import jax
import jax.numpy as jnp
from jax import lax
from jax.experimental import pallas as pl
from jax.experimental.pallas import tpu as pltpu
from jax.experimental.pallas import tpu_sc as plsc

_K = 1024
_D = 64
_BETA = 0.25
_KC = 256

_NC = 2
_NS = 16
_NW = _NC * _NS
_CH = 128
_NBUF = 3


def _argmin_body(xt_ref, xsq_ref, w_ref, wsq_ref, kio_ref, wt_ref, kcol_ref,
                 idx_ref, qt_ref, loss_ref, acc_ref, *, n_tokens, r):
    x2t = xt_ref[0] * 2.0
    xsq_row = xsq_ref[0]

    big = float(2 * _K)
    run_min8 = jnp.full((8, r), jnp.inf, dtype=jnp.float32)
    run_k8 = jnp.zeros((8, r), dtype=jnp.float32)
    for c in range(_K // _KC):
        wc = w_ref[c * _KC:(c + 1) * _KC, :]
        wsq_c = wsq_ref[c * _KC:(c + 1) * _KC, :]
        mm2 = lax.dot_general(wc, x2t, (((1,), (0,)), ((), ())),
                              preferred_element_type=jnp.float32)
        dist = (xsq_row + wsq_c) - mm2
        d3 = dist.reshape(_KC // 8, 8, r)
        cmin8 = jnp.min(d3, axis=0)
        kio_c = kio_ref[c * (_KC // 8):(c + 1) * (_KC // 8), :][:, :, None]
        ck8 = jnp.min(jnp.where(d3 == cmin8[None], kio_c, big), axis=0)
        better = cmin8 < run_min8
        run_k8 = jnp.where(better, ck8, run_k8)
        run_min8 = jnp.where(better, cmin8, run_min8)

    gmin = jnp.min(run_min8, axis=0)
    kbest = jnp.min(jnp.where(run_min8 == gmin[None, :], run_k8, big), axis=0)
    idx_ref[...] = kbest.astype(jnp.int32)[None, None, :]

    q = jnp.zeros((_D, r), dtype=jnp.float32)
    for c in range(_K // _KC):
        kcol_c = kcol_ref[c * _KC:(c + 1) * _KC, :]
        oh = (kcol_c == kbest[None, :]).astype(jnp.float32)
        wt_c = wt_ref[:, c * _KC:(c + 1) * _KC]
        q = q + lax.dot_general(wt_c, oh, (((1,), (0,)), ((), ())),
                                preferred_element_type=jnp.float32)
    qt_ref[...] = q[None]

    @pl.when(pl.program_id(0) == 0)
    def _init():
        acc_ref[0, 0] = 0.0

    acc_ref[0, 0] += jnp.sum(gmin)
    scale = (1.0 + _BETA) / (n_tokens * _D)
    loss_ref[...] = jnp.broadcast_to(acc_ref[0, 0] * scale, (1, 1))


def _gather_body(w_hbm, idx_hbm, out_hbm, idx_v, rows, gsems, wsems, *, chunks):
    wid = lax.axis_index("s") * _NC + lax.axis_index("c")
    base = wid * (chunks * _CH)
    pltpu.sync_copy(idx_hbm.at[pl.ds(base, chunks * _CH)], idx_v)
    gcps = [None] * _NBUF
    wcps = [None] * _NBUF
    for j in range(chunks):
        b = j % _NBUF
        if wcps[b] is not None:
            wcps[b].wait()
        gcps[b] = pltpu.async_copy(
            w_hbm.at[idx_v.at[pl.ds(j * _CH, _CH)]], rows[b], gsems[b])
        if j >= _NBUF - 1:
            jw = j - (_NBUF - 1)
            bw = jw % _NBUF
            gcps[bw].wait()
            wcps[bw] = pltpu.async_copy(
                rows[bw], out_hbm.at[pl.ds(base + jw * _CH, _CH)], wsems[bw])
    for jw in range(max(chunks - _NBUF + 1, 0), chunks):
        bw = jw % _NBUF
        gcps[bw].wait()
        wcps[bw] = pltpu.async_copy(
            rows[bw], out_hbm.at[pl.ds(base + jw * _CH, _CH)], wsems[bw])
    for b in range(_NBUF):
        if wcps[b] is not None:
            wcps[b].wait()


def kernel(latents, W):
    lat = latents.reshape(-1, latents.shape[-2], _D)
    flat = lat.reshape(-1, _D)
    n = flat.shape[0]
    nb, node = lat.shape[0], lat.shape[1]
    latT = jnp.transpose(lat, (0, 2, 1))
    xsq = jnp.sum(flat ** 2, axis=1).reshape(nb, 1, node)
    wsq = jnp.sum(W ** 2, axis=1)[:, None]
    kio = jnp.arange(_K, dtype=jnp.float32).reshape(_K // 8, 8)

    grid = (nb,)
    idx, qt, loss = pl.pallas_call(
        functools.partial(_argmin_body, n_tokens=n, r=node),
        grid=grid,
        in_specs=[
            pl.BlockSpec((1, _D, node), lambda i: (i, 0, 0)),
            pl.BlockSpec((1, 1, node), lambda i: (i, 0, 0)),
            pl.BlockSpec((_K, _D), lambda i: (0, 0)),
            pl.BlockSpec((_K, 1), lambda i: (0, 0)),
            pl.BlockSpec((_K // 8, 8), lambda i: (0, 0)),
            pl.BlockSpec((_D, _K), lambda i: (0, 0)),
            pl.BlockSpec((_K, 1), lambda i: (0, 0)),
        ],
        out_specs=[
            pl.BlockSpec((1, 1, node), lambda i: (i, 0, 0)),
            pl.BlockSpec((1, _D, node), lambda i: (i, 0, 0)),
            pl.BlockSpec((1, 1), lambda i: (0, 0)),
        ],
        out_shape=[
            jax.ShapeDtypeStruct((nb, 1, node), jnp.int32),
            jax.ShapeDtypeStruct((nb, _D, node), jnp.float32),
            jax.ShapeDtypeStruct((1, 1), jnp.float32),
        ],
        scratch_shapes=[pltpu.SMEM((1, 1), jnp.float32)],
    )(latT, xsq, W, wsq, kio, W.T,
      jnp.arange(_K, dtype=jnp.float32)[:, None])

    quantized_st = qt.transpose(0, 2, 1).reshape(lat.shape)
    vq_loss = loss[0, 0]
    return (quantized_st, vq_loss)
